# Initial kernel scaffold; baseline (speedup 1.0000x reference)
import jax
import jax.numpy as jnp
from jax import lax
from jax.experimental import pallas as pl
from jax.experimental.pallas import tpu as pltpu
from jax.experimental.pallas import tpu_sc as plsc

N_NODES = 100000
N_EDGES = 1600000
D_FEAT = 128
SD = 32
HF = 16
NG = 512
ROUNDS_N = 4

NC = 2
NS = 16
CH = 128
NB = 8

EPG = NS * CH * NB
NGRP = -(-N_EDGES // EPG)
EP = NGRP * EPG
EPT = EP // NS
NPAD = 100352
ZPT = NPAD // NS
ZB = 448

NB2 = 7
NGRP2 = 7
GPAD = 520

_EDGE_SCRATCH = (
    [pltpu.VMEM_SHARED((NPAD, HF), jnp.float32)]
    + [pltpu.VMEM((ZB, HF), jnp.float32)]
    + [pltpu.VMEM((CH,), jnp.int32) for _ in range(NB)]
    + [pltpu.VMEM((CH,), jnp.int32) for _ in range(NB)]
    + [pltpu.VMEM((CH, HF), jnp.float32) for _ in range(NB)]
    + [pltpu.SemaphoreType.DMA]
    + [pltpu.SemaphoreType.DMA for _ in range(3 * NB)]
)


def _edge_body(msg2, srcs, dstp, out, *scr):
    acc = scr[0]
    zbuf = scr[1]
    sidx = scr[2:2 + NB]
    didx = scr[2 + NB:2 + 2 * NB]
    gbuf = scr[2 + 2 * NB:2 + 3 * NB]
    zsem = scr[2 + 3 * NB]
    isem = scr[3 + 3 * NB:3 + 4 * NB]
    gsem = scr[3 + 4 * NB:3 + 5 * NB]
    ssem = scr[3 + 5 * NB:3 + 6 * NB]

    c = lax.axis_index("c")
    s = lax.axis_index("s")

    def zfill(i, carry):
        zbuf[i, :] = jnp.zeros((HF,), jnp.float32)
        return carry
    lax.fori_loop(0, ZB, zfill, 0)
    z0 = s * ZPT
    for k in range(ZPT // ZB):
        pltpu.async_copy(zbuf, acc.at[pl.ds(z0 + k * ZB, ZB)], zsem)
    for k in range(ZPT // ZB):
        pltpu.make_async_copy(zbuf, acc.at[pl.ds(z0 + k * ZB, ZB)], zsem).wait()
    plsc.subcore_barrier()

    ebase = s * EPT

    def issue_idx(off, b):
        pltpu.async_copy(srcs.at[c, pl.ds(off, CH)], sidx[b], isem[b])
        pltpu.async_copy(dstp.at[pl.ds(off, CH)], didx[b], isem[b])

    for b in range(NB):
        issue_idx(ebase + b * CH, b)

    def group(g, carry):
        for b in range(NB):
            pltpu.make_async_copy(srcs.at[c, pl.ds(0, CH)], sidx[b], isem[b]).wait()
            pltpu.make_async_copy(dstp.at[pl.ds(0, CH)], didx[b], isem[b]).wait()
            pltpu.async_copy(msg2.at[sidx[b]], gbuf[b], gsem[b])
        for b in range(NB):
            pltpu.make_async_copy(msg2.at[sidx[b]], gbuf[b], gsem[b]).wait()
            pltpu.async_copy(gbuf[b], acc.at[didx[b]], ssem[b], add=True)
        for b in range(NB):
            pltpu.make_async_copy(gbuf[b], acc.at[didx[b]], ssem[b]).wait()

            @pl.when(g + 1 < NGRP)
            def _(b=b):
                issue_idx(ebase + ((g + 1) * NB + b) * CH, b)
        return carry

    lax.fori_loop(0, NGRP, group, 0)
    plsc.subcore_barrier()

    pltpu.sync_copy(acc.at[pl.ds(z0, ZPT)], out.at[c, pl.ds(z0, ZPT)])


_edge_kernel = pl.kernel(
    _edge_body,
    out_type=jax.ShapeDtypeStruct((NC, NPAD, HF), jnp.float32),
    mesh=plsc.VectorSubcoreMesh(
        core_axis_name="c", subcore_axis_name="s", num_cores=NC,
        num_subcores=NS,
    ),
    scratch_types=_EDGE_SCRATCH,
)


_BATCH_SCRATCH = (
    [pltpu.VMEM_SHARED((GPAD, HF), jnp.float32)]
    + [pltpu.VMEM((GPAD, HF), jnp.float32)]
    + [pltpu.VMEM((CH,), jnp.int32) for _ in range(NB2)]
    + [pltpu.VMEM((CH, HF), jnp.float32) for _ in range(NB2)]
    + [pltpu.SemaphoreType.DMA]
    + [pltpu.SemaphoreType.DMA for _ in range(2 * NB2)]
)


def _batch_body(st2, bidx_hbm, outg, *scr):
    acc = scr[0]
    zbuf = scr[1]
    bidx = scr[2:2 + NB2]
    sbuf = scr[2 + NB2:2 + 2 * NB2]
    zsem = scr[2 + 2 * NB2]
    isem = scr[3 + 2 * NB2:3 + 3 * NB2]
    ssem = scr[3 + 3 * NB2:3 + 4 * NB2]

    c = lax.axis_index("c")
    s = lax.axis_index("s")

    @pl.when(s == 0)
    def _():
        def zfill(i, carry):
            zbuf[i, :] = jnp.zeros((HF,), jnp.float32)
            return carry
        lax.fori_loop(0, GPAD, zfill, 0)
        pltpu.async_copy(zbuf, acc, zsem)
        pltpu.make_async_copy(zbuf, acc, zsem).wait()
    plsc.subcore_barrier()

    rbase = s * ZPT

    def issue_idx(off, b):
        pltpu.async_copy(st2.at[c, pl.ds(off, CH)], sbuf[b], isem[b])
        pltpu.async_copy(bidx_hbm.at[pl.ds(off, CH)], bidx[b], isem[b])

    for b in range(NB2):
        issue_idx(rbase + b * CH, b)

    def group(g, carry):
        for b in range(NB2):
            pltpu.make_async_copy(st2.at[c, pl.ds(0, CH)], sbuf[b], isem[b]).wait()
            pltpu.make_async_copy(bidx_hbm.at[pl.ds(0, CH)], bidx[b], isem[b]).wait()
            pltpu.async_copy(sbuf[b], acc.at[bidx[b]], ssem[b], add=True)
        for b in range(NB2):
            pltpu.make_async_copy(sbuf[b], acc.at[bidx[b]], ssem[b]).wait()

            @pl.when(g + 1 < NGRP2)
            def _(b=b):
                issue_idx(rbase + ((g + 1) * NB2 + b) * CH, b)
        return carry

    lax.fori_loop(0, NGRP2, group, 0)
    plsc.subcore_barrier()

    @pl.when(s == 0)
    def _():
        pltpu.sync_copy(acc.at[pl.ds(0, NG)], outg.at[c])


_batch_kernel = pl.kernel(
    _batch_body,
    out_type=jax.ShapeDtypeStruct((NC, NG, HF), jnp.float32),
    mesh=plsc.VectorSubcoreMesh(
        core_axis_name="c", subcore_axis_name="s", num_cores=NC,
        num_subcores=NS,
    ),
    scratch_types=_BATCH_SCRATCH,
)


RB = 2000
NBLK = N_NODES // RB

_P = jax.lax.Precision.HIGHEST


def _tc_in_body(x_ref, wi, bi, mw, mb, st_ref, msg_ref):
    st = jnp.maximum(
        jnp.dot(x_ref[...], wi[...], preferred_element_type=jnp.float32,
                precision=_P) + bi[...], 0.0)
    st_ref[...] = st
    m = jnp.maximum(
        jnp.dot(st, mw[...], preferred_element_type=jnp.float32,
                precision=_P) + mb[...], 0.0)
    msg_ref[0] = m[:, :HF]
    msg_ref[1] = m[:, HF:]


def _tc_round_body(agg_ref, st_ref, uw, ub, mw, mb, st_out, msg_ref):
    a = jnp.concatenate([agg_ref[0], agg_ref[1]], axis=-1)
    st = st_ref[...] + jnp.maximum(
        jnp.dot(a, uw[...], preferred_element_type=jnp.float32,
                precision=_P) + ub[...], 0.0)
    st_out[...] = st
    m = jnp.maximum(
        jnp.dot(st, mw[...], preferred_element_type=jnp.float32,
                precision=_P) + mb[...], 0.0)
    msg_ref[0] = m[:, :HF]
    msg_ref[1] = m[:, HF:]


def _tc_last_body(agg_ref, st_ref, uw, ub, st2_ref):
    a = jnp.concatenate([agg_ref[0], agg_ref[1]], axis=-1)
    st = st_ref[...] + jnp.maximum(
        jnp.dot(a, uw[...], preferred_element_type=jnp.float32,
                precision=_P) + ub[...], 0.0)
    st2_ref[0] = st[:, :HF]
    st2_ref[1] = st[:, HF:]


def _tc_final_body(g0, g1, wo, bo, out_ref):
    gs = jnp.concatenate([g0[...], g1[...]], axis=-1)
    o = jnp.dot(gs, wo[...], preferred_element_type=jnp.float32,
                precision=_P) + bo[...]
    out_ref[...] = jnp.concatenate([o[:, :SD], jnp.exp(o[:, SD:])], axis=-1)


def _full(shape):
    return pl.BlockSpec(shape, lambda i: (0,) * len(shape))


_tc_in = pl.pallas_call(
    _tc_in_body,
    grid=(NBLK,),
    in_specs=[
        pl.BlockSpec((RB, D_FEAT), lambda i: (i, 0)),
        _full((D_FEAT, SD)), _full((1, SD)), _full((SD, SD)), _full((1, SD)),
    ],
    out_specs=[
        pl.BlockSpec((RB, SD), lambda i: (i, 0)),
        pl.BlockSpec((NC, RB, HF), lambda i: (0, i, 0)),
    ],
    out_shape=[
        jax.ShapeDtypeStruct((N_NODES, SD), jnp.float32),
        jax.ShapeDtypeStruct((NC, N_NODES, HF), jnp.float32),
    ],
)

_tc_round = pl.pallas_call(
    _tc_round_body,
    grid=(NBLK,),
    in_specs=[
        pl.BlockSpec((NC, RB, HF), lambda i: (0, i, 0)),
        pl.BlockSpec((RB, SD), lambda i: (i, 0)),
        _full((SD, SD)), _full((1, SD)), _full((SD, SD)), _full((1, SD)),
    ],
    out_specs=[
        pl.BlockSpec((RB, SD), lambda i: (i, 0)),
        pl.BlockSpec((NC, RB, HF), lambda i: (0, i, 0)),
    ],
    out_shape=[
        jax.ShapeDtypeStruct((N_NODES, SD), jnp.float32),
        jax.ShapeDtypeStruct((NC, N_NODES, HF), jnp.float32),
    ],
)

_tc_last = pl.pallas_call(
    _tc_last_body,
    grid=(NBLK,),
    in_specs=[
        pl.BlockSpec((NC, RB, HF), lambda i: (0, i, 0)),
        pl.BlockSpec((RB, SD), lambda i: (i, 0)),
        _full((SD, SD)), _full((1, SD)),
    ],
    out_specs=pl.BlockSpec((NC, RB, HF), lambda i: (0, i, 0)),
    out_shape=jax.ShapeDtypeStruct((NC, NPAD, HF), jnp.float32),
)

_tc_final = pl.pallas_call(
    _tc_final_body,
    grid=(1,),
    in_specs=[
        _full((NG, HF)), _full((NG, HF)),
        _full((SD, 2 * SD)), _full((1, 2 * SD)),
    ],
    out_specs=_full((NG, 2 * SD)),
    out_shape=jax.ShapeDtypeStruct((NG, 2 * SD), jnp.float32),
)


@jax.jit
def kernel(x, edge_index, batch, W_in, b_in, msg_W, msg_b, upd_W, upd_b,
           W_out, b_out):
    src = edge_index[0].astype(jnp.int32)
    dst = edge_index[1].astype(jnp.int32)
    pad = EP - N_EDGES
    srcp = jnp.concatenate([src, jnp.zeros((pad,), jnp.int32)])
    srcs = jnp.stack([srcp, srcp + N_NODES])
    dstp = jnp.concatenate([dst, jnp.full((pad,), N_NODES, jnp.int32)])
    bpad = jnp.concatenate(
        [batch.astype(jnp.int32), jnp.full((NPAD - N_NODES,), NG, jnp.int32)])

    bi = b_in.reshape(1, SD)
    mb = msg_b.reshape(ROUNDS_N, 1, SD)
    ub = upd_b.reshape(ROUNDS_N, 1, SD)

    state, msg2 = _tc_in(x, W_in, bi, msg_W[0], mb[0])
    st2 = None
    for r in range(ROUNDS_N):
        agg = _edge_kernel(msg2.reshape(NC * N_NODES, HF), srcs, dstp)
        if r < ROUNDS_N - 1:
            state, msg2 = _tc_round(agg, state, upd_W[r], ub[r],
                                    msg_W[r + 1], mb[r + 1])
        else:
            st2 = _tc_last(agg, state, upd_W[r], ub[r])
    gs2 = _batch_kernel(st2, bpad)
    out = _tc_final(gs2[0], gs2[1], W_out, b_out.reshape(1, 2 * SD))
    return out


# speedup vs baseline: 11.6906x; 11.6906x over previous
import jax
import jax.numpy as jnp
from jax import lax
from jax.experimental import pallas as pl
from jax.experimental.pallas import tpu as pltpu
from jax.experimental.pallas import tpu_sc as plsc

N_NODES = 100000
N_EDGES = 1600000
D_FEAT = 128
SD = 32
HF = 16
NG = 512
ROUNDS_N = 4

NC = 2
NS = 16
CH = 128
NB = 8

EPG = NS * CH * NB
NGRP = -(-N_EDGES // EPG)
EP = NGRP * EPG
EPT = EP // NS
NPAD = 100352
ZPT = NPAD // NS
ZB = 448

NB2 = 7
NGRP2 = 7
GPAD = 520

_EDGE_SCRATCH = (
    [pltpu.VMEM_SHARED((NPAD, HF), jnp.float32)]
    + [pltpu.VMEM((ZB, HF), jnp.float32)]
    + [pltpu.VMEM((CH,), jnp.int32) for _ in range(NB)]
    + [pltpu.VMEM((CH,), jnp.int32) for _ in range(NB)]
    + [pltpu.VMEM((CH, HF), jnp.float32) for _ in range(NB)]
    + [pltpu.SemaphoreType.DMA]
    + [pltpu.SemaphoreType.DMA for _ in range(3 * NB)]
)


def _edge_body(msg2, srcs, dstp, out, *scr):
    acc = scr[0]
    zbuf = scr[1]
    sidx = scr[2:2 + NB]
    didx = scr[2 + NB:2 + 2 * NB]
    gbuf = scr[2 + 2 * NB:2 + 3 * NB]
    zsem = scr[2 + 3 * NB]
    isem = scr[3 + 3 * NB:3 + 4 * NB]
    gsem = scr[3 + 4 * NB:3 + 5 * NB]
    ssem = scr[3 + 5 * NB:3 + 6 * NB]

    c = lax.axis_index("c")
    s = lax.axis_index("s")

    def zfill(i, carry):
        zbuf[i, :] = jnp.zeros((HF,), jnp.float32)
        return carry
    lax.fori_loop(0, ZB, zfill, 0)
    z0 = s * ZPT
    for k in range(ZPT // ZB):
        pltpu.async_copy(zbuf, acc.at[pl.ds(z0 + k * ZB, ZB)], zsem)
    for k in range(ZPT // ZB):
        pltpu.make_async_copy(zbuf, acc.at[pl.ds(z0 + k * ZB, ZB)], zsem).wait()
    plsc.subcore_barrier()

    ebase = s * EPT

    def issue_idx(off, b):
        pltpu.async_copy(srcs.at[c, pl.ds(off, CH)], sidx[b], isem[b])
        pltpu.async_copy(dstp.at[pl.ds(off, CH)], didx[b], isem[b])

    for b in range(NB):
        issue_idx(ebase + b * CH, b)

    def group(g, carry):
        for b in range(NB):
            pltpu.make_async_copy(srcs.at[c, pl.ds(0, CH)], sidx[b], isem[b]).wait()
            pltpu.make_async_copy(dstp.at[pl.ds(0, CH)], didx[b], isem[b]).wait()
            pltpu.async_copy(msg2.at[sidx[b]], gbuf[b], gsem[b])
        for b in range(NB):
            pltpu.make_async_copy(msg2.at[sidx[b]], gbuf[b], gsem[b]).wait()
            pltpu.async_copy(gbuf[b], acc.at[didx[b]], ssem[b], add=True)
        for b in range(NB):
            pltpu.make_async_copy(gbuf[b], acc.at[didx[b]], ssem[b]).wait()

            @pl.when(g + 1 < NGRP)
            def _(b=b):
                issue_idx(ebase + ((g + 1) * NB + b) * CH, b)
        return carry

    lax.fori_loop(0, NGRP, group, 0)
    plsc.subcore_barrier()

    pltpu.sync_copy(acc.at[pl.ds(z0, ZPT)], out.at[c, pl.ds(z0, ZPT)])


_edge_kernel = pl.kernel(
    _edge_body,
    out_type=jax.ShapeDtypeStruct((NC, NPAD, HF), jnp.float32),
    mesh=plsc.VectorSubcoreMesh(
        core_axis_name="c", subcore_axis_name="s", num_cores=NC,
        num_subcores=NS,
    ),
    scratch_types=_EDGE_SCRATCH,
    compiler_params=pltpu.CompilerParams(use_tc_tiling_on_sc=False),
)


_BATCH_SCRATCH = (
    [pltpu.VMEM_SHARED((GPAD, HF), jnp.float32)]
    + [pltpu.VMEM((GPAD, HF), jnp.float32)]
    + [pltpu.VMEM((CH,), jnp.int32) for _ in range(NB2)]
    + [pltpu.VMEM((CH, HF), jnp.float32) for _ in range(NB2)]
    + [pltpu.SemaphoreType.DMA]
    + [pltpu.SemaphoreType.DMA for _ in range(2 * NB2)]
)


def _batch_body(st2, bidx_hbm, outg, *scr):
    acc = scr[0]
    zbuf = scr[1]
    bidx = scr[2:2 + NB2]
    sbuf = scr[2 + NB2:2 + 2 * NB2]
    zsem = scr[2 + 2 * NB2]
    isem = scr[3 + 2 * NB2:3 + 3 * NB2]
    ssem = scr[3 + 3 * NB2:3 + 4 * NB2]

    c = lax.axis_index("c")
    s = lax.axis_index("s")

    @pl.when(s == 0)
    def _():
        def zfill(i, carry):
            zbuf[i, :] = jnp.zeros((HF,), jnp.float32)
            return carry
        lax.fori_loop(0, GPAD, zfill, 0)
        pltpu.async_copy(zbuf, acc, zsem)
        pltpu.make_async_copy(zbuf, acc, zsem).wait()
    plsc.subcore_barrier()

    rbase = s * ZPT

    def issue_idx(off, b):
        pltpu.async_copy(st2.at[c, pl.ds(off, CH)], sbuf[b], isem[b])
        pltpu.async_copy(bidx_hbm.at[pl.ds(off, CH)], bidx[b], isem[b])

    for b in range(NB2):
        issue_idx(rbase + b * CH, b)

    def group(g, carry):
        for b in range(NB2):
            pltpu.make_async_copy(st2.at[c, pl.ds(0, CH)], sbuf[b], isem[b]).wait()
            pltpu.make_async_copy(bidx_hbm.at[pl.ds(0, CH)], bidx[b], isem[b]).wait()
            pltpu.async_copy(sbuf[b], acc.at[bidx[b]], ssem[b], add=True)
        for b in range(NB2):
            pltpu.make_async_copy(sbuf[b], acc.at[bidx[b]], ssem[b]).wait()

            @pl.when(g + 1 < NGRP2)
            def _(b=b):
                issue_idx(rbase + ((g + 1) * NB2 + b) * CH, b)
        return carry

    lax.fori_loop(0, NGRP2, group, 0)
    plsc.subcore_barrier()

    @pl.when(s == 0)
    def _():
        pltpu.sync_copy(acc.at[pl.ds(0, NG)], outg.at[c])


_batch_kernel = pl.kernel(
    _batch_body,
    out_type=jax.ShapeDtypeStruct((NC, NG, HF), jnp.float32),
    mesh=plsc.VectorSubcoreMesh(
        core_axis_name="c", subcore_axis_name="s", num_cores=NC,
        num_subcores=NS,
    ),
    scratch_types=_BATCH_SCRATCH,
    compiler_params=pltpu.CompilerParams(use_tc_tiling_on_sc=False),
)


RB = 2000
NBLK = N_NODES // RB

_P = jax.lax.Precision.HIGHEST


def _tc_in_body(x_ref, wi, bi, mw, mb, st_ref, msg_ref):
    st = jnp.maximum(
        jnp.dot(x_ref[...], wi[...], preferred_element_type=jnp.float32,
                precision=_P) + bi[...], 0.0)
    st_ref[...] = st
    m = jnp.maximum(
        jnp.dot(st, mw[...], preferred_element_type=jnp.float32,
                precision=_P) + mb[...], 0.0)
    msg_ref[0] = m[:, :HF]
    msg_ref[1] = m[:, HF:]


def _tc_round_body(agg_ref, st_ref, uw, ub, mw, mb, st_out, msg_ref):
    a = jnp.concatenate([agg_ref[0], agg_ref[1]], axis=-1)
    st = st_ref[...] + jnp.maximum(
        jnp.dot(a, uw[...], preferred_element_type=jnp.float32,
                precision=_P) + ub[...], 0.0)
    st_out[...] = st
    m = jnp.maximum(
        jnp.dot(st, mw[...], preferred_element_type=jnp.float32,
                precision=_P) + mb[...], 0.0)
    msg_ref[0] = m[:, :HF]
    msg_ref[1] = m[:, HF:]


def _tc_last_body(agg_ref, st_ref, uw, ub, st2_ref):
    a = jnp.concatenate([agg_ref[0], agg_ref[1]], axis=-1)
    st = st_ref[...] + jnp.maximum(
        jnp.dot(a, uw[...], preferred_element_type=jnp.float32,
                precision=_P) + ub[...], 0.0)
    st2_ref[0] = st[:, :HF]
    st2_ref[1] = st[:, HF:]


def _tc_final_body(g0, g1, wo, bo, out_ref):
    gs = jnp.concatenate([g0[...], g1[...]], axis=-1)
    o = jnp.dot(gs, wo[...], preferred_element_type=jnp.float32,
                precision=_P) + bo[...]
    out_ref[...] = jnp.concatenate([o[:, :SD], jnp.exp(o[:, SD:])], axis=-1)


def _full(shape):
    return pl.BlockSpec(shape, lambda i: (0,) * len(shape))


_tc_in = pl.pallas_call(
    _tc_in_body,
    grid=(NBLK,),
    in_specs=[
        pl.BlockSpec((RB, D_FEAT), lambda i: (i, 0)),
        _full((D_FEAT, SD)), _full((1, SD)), _full((SD, SD)), _full((1, SD)),
    ],
    out_specs=[
        pl.BlockSpec((RB, SD), lambda i: (i, 0)),
        pl.BlockSpec((NC, RB, HF), lambda i: (0, i, 0)),
    ],
    out_shape=[
        jax.ShapeDtypeStruct((N_NODES, SD), jnp.float32),
        jax.ShapeDtypeStruct((NC, N_NODES, HF), jnp.float32),
    ],
)

_tc_round = pl.pallas_call(
    _tc_round_body,
    grid=(NBLK,),
    in_specs=[
        pl.BlockSpec((NC, RB, HF), lambda i: (0, i, 0)),
        pl.BlockSpec((RB, SD), lambda i: (i, 0)),
        _full((SD, SD)), _full((1, SD)), _full((SD, SD)), _full((1, SD)),
    ],
    out_specs=[
        pl.BlockSpec((RB, SD), lambda i: (i, 0)),
        pl.BlockSpec((NC, RB, HF), lambda i: (0, i, 0)),
    ],
    out_shape=[
        jax.ShapeDtypeStruct((N_NODES, SD), jnp.float32),
        jax.ShapeDtypeStruct((NC, N_NODES, HF), jnp.float32),
    ],
)

_tc_last = pl.pallas_call(
    _tc_last_body,
    grid=(NBLK,),
    in_specs=[
        pl.BlockSpec((NC, RB, HF), lambda i: (0, i, 0)),
        pl.BlockSpec((RB, SD), lambda i: (i, 0)),
        _full((SD, SD)), _full((1, SD)),
    ],
    out_specs=pl.BlockSpec((NC, RB, HF), lambda i: (0, i, 0)),
    out_shape=jax.ShapeDtypeStruct((NC, NPAD, HF), jnp.float32),
)

_tc_final = pl.pallas_call(
    _tc_final_body,
    grid=(1,),
    in_specs=[
        _full((NG, HF)), _full((NG, HF)),
        _full((SD, 2 * SD)), _full((1, 2 * SD)),
    ],
    out_specs=_full((NG, 2 * SD)),
    out_shape=jax.ShapeDtypeStruct((NG, 2 * SD), jnp.float32),
)


@jax.jit
def kernel(x, edge_index, batch, W_in, b_in, msg_W, msg_b, upd_W, upd_b,
           W_out, b_out):
    src = edge_index[0].astype(jnp.int32)
    dst = edge_index[1].astype(jnp.int32)
    pad = EP - N_EDGES
    srcp = jnp.concatenate([src, jnp.zeros((pad,), jnp.int32)])
    srcs = jnp.stack([srcp, srcp + N_NODES])
    dstp = jnp.concatenate([dst, jnp.full((pad,), N_NODES, jnp.int32)])
    bpad = jnp.concatenate(
        [batch.astype(jnp.int32), jnp.full((NPAD - N_NODES,), NG, jnp.int32)])

    bi = b_in.reshape(1, SD)
    mb = msg_b.reshape(ROUNDS_N, 1, SD)
    ub = upd_b.reshape(ROUNDS_N, 1, SD)

    state, msg2 = _tc_in(x, W_in, bi, msg_W[0], mb[0])
    st2 = None
    for r in range(ROUNDS_N):
        agg = _edge_kernel(msg2.reshape(NC * N_NODES, HF), srcs, dstp)
        if r < ROUNDS_N - 1:
            state, msg2 = _tc_round(agg, state, upd_W[r], ub[r],
                                    msg_W[r + 1], mb[r + 1])
        else:
            st2 = _tc_last(agg, state, upd_W[r], ub[r])
    gs2 = _batch_kernel(st2, bpad)
    out = _tc_final(gs2[0], gs2[1], W_out, b_out.reshape(1, 2 * SD))
    return out


# speedup vs baseline: 18.0031x; 1.5400x over previous
import jax
import jax.numpy as jnp
from jax import lax
from jax.experimental import pallas as pl
from jax.experimental.pallas import tpu as pltpu
from jax.experimental.pallas import tpu_sc as plsc

N_NODES = 100000
N_EDGES = 1600000
D_FEAT = 128
SD = 32
HF = 16
NG = 512
ROUNDS_N = 4

NC = 2
NS = 16
CH = 128
NB = 8

EPG = NS * CH * NB
NGRP = -(-N_EDGES // EPG)
EP = NGRP * EPG
EPT = EP // NS
NPAD = 100352
ZPT = NPAD // NS
ZB = 448

NB2 = 7
NGRP2 = 7
GPAD = 520

_EDGE_SCRATCH = (
    [pltpu.VMEM_SHARED((NPAD, HF), jnp.float32)]
    + [pltpu.VMEM((ZB, HF), jnp.float32)]
    + [pltpu.VMEM((CH,), jnp.int32) for _ in range(NB)]
    + [pltpu.VMEM((CH,), jnp.int32) for _ in range(NB)]
    + [pltpu.VMEM((CH, HF), jnp.float32) for _ in range(NB)]
    + [pltpu.SemaphoreType.DMA]
    + [pltpu.SemaphoreType.DMA for _ in range(3 * NB)]
)


def _edge_body(msg2, srcs, dstp, out, *scr):
    acc = scr[0]
    zbuf = scr[1]
    sidx = scr[2:2 + NB]
    didx = scr[2 + NB:2 + 2 * NB]
    gbuf = scr[2 + 2 * NB:2 + 3 * NB]
    zsem = scr[2 + 3 * NB]
    isem = scr[3 + 3 * NB:3 + 4 * NB]
    gsem = scr[3 + 4 * NB:3 + 5 * NB]
    ssem = scr[3 + 5 * NB:3 + 6 * NB]

    c = lax.axis_index("c")
    s = lax.axis_index("s")

    def zfill(i, carry):
        zbuf[i, :] = jnp.zeros((HF,), jnp.float32)
        return carry
    lax.fori_loop(0, ZB, zfill, 0)
    z0 = s * ZPT
    for k in range(ZPT // ZB):
        pltpu.async_copy(zbuf, acc.at[pl.ds(z0 + k * ZB, ZB)], zsem)
    for k in range(ZPT // ZB):
        pltpu.make_async_copy(zbuf, acc.at[pl.ds(z0 + k * ZB, ZB)], zsem).wait()
    plsc.subcore_barrier()

    ebase = s * EPT

    def issue_idx(off, b):
        pltpu.async_copy(srcs.at[pl.ds(c * EP + off, CH)], sidx[b], isem[b])
        pltpu.async_copy(dstp.at[pl.ds(off, CH)], didx[b], isem[b])

    for b in range(NB):
        issue_idx(ebase + b * CH, b)

    def group(g, carry):
        for b in range(NB):
            pltpu.make_async_copy(srcs.at[pl.ds(0, CH)], sidx[b], isem[b]).wait()
            pltpu.make_async_copy(dstp.at[pl.ds(0, CH)], didx[b], isem[b]).wait()
            pltpu.async_copy(msg2.at[sidx[b]], gbuf[b], gsem[b])
        for b in range(NB):
            pltpu.make_async_copy(msg2.at[sidx[b]], gbuf[b], gsem[b]).wait()
            pltpu.async_copy(gbuf[b], acc.at[didx[b]], ssem[b], add=True)
        for b in range(NB):
            pltpu.make_async_copy(gbuf[b], acc.at[didx[b]], ssem[b]).wait()

            @pl.when(g + 1 < NGRP)
            def _(b=b):
                issue_idx(ebase + ((g + 1) * NB + b) * CH, b)
        return carry

    lax.fori_loop(0, NGRP, group, 0)
    plsc.subcore_barrier()

    pltpu.sync_copy(acc.at[pl.ds(z0, ZPT)], out.at[c, pl.ds(z0, ZPT)])


_edge_kernel = pl.kernel(
    _edge_body,
    out_type=jax.ShapeDtypeStruct((NC, NPAD, HF), jnp.float32),
    mesh=plsc.VectorSubcoreMesh(
        core_axis_name="c", subcore_axis_name="s", num_cores=NC,
        num_subcores=NS,
    ),
    scratch_types=_EDGE_SCRATCH,
    compiler_params=pltpu.CompilerParams(use_tc_tiling_on_sc=False),
)


_BATCH_SCRATCH = (
    [pltpu.VMEM_SHARED((GPAD, HF), jnp.float32)]
    + [pltpu.VMEM((GPAD, HF), jnp.float32)]
    + [pltpu.VMEM((CH,), jnp.int32) for _ in range(NB2)]
    + [pltpu.VMEM((CH, HF), jnp.float32) for _ in range(NB2)]
    + [pltpu.SemaphoreType.DMA]
    + [pltpu.SemaphoreType.DMA for _ in range(2 * NB2)]
)


def _batch_body(st2, bidx_hbm, outg, *scr):
    acc = scr[0]
    zbuf = scr[1]
    bidx = scr[2:2 + NB2]
    sbuf = scr[2 + NB2:2 + 2 * NB2]
    zsem = scr[2 + 2 * NB2]
    isem = scr[3 + 2 * NB2:3 + 3 * NB2]
    ssem = scr[3 + 3 * NB2:3 + 4 * NB2]

    c = lax.axis_index("c")
    s = lax.axis_index("s")

    @pl.when(s == 0)
    def _():
        def zfill(i, carry):
            zbuf[i, :] = jnp.zeros((HF,), jnp.float32)
            return carry
        lax.fori_loop(0, GPAD, zfill, 0)
        pltpu.async_copy(zbuf, acc, zsem)
        pltpu.make_async_copy(zbuf, acc, zsem).wait()
    plsc.subcore_barrier()

    rbase = s * ZPT

    def issue_idx(off, b):
        pltpu.async_copy(st2.at[c, pl.ds(off, CH)], sbuf[b], isem[b])
        pltpu.async_copy(bidx_hbm.at[pl.ds(off, CH)], bidx[b], isem[b])

    for b in range(NB2):
        issue_idx(rbase + b * CH, b)

    def group(g, carry):
        for b in range(NB2):
            pltpu.make_async_copy(st2.at[c, pl.ds(0, CH)], sbuf[b], isem[b]).wait()
            pltpu.make_async_copy(bidx_hbm.at[pl.ds(0, CH)], bidx[b], isem[b]).wait()
            pltpu.async_copy(sbuf[b], acc.at[bidx[b]], ssem[b], add=True)
        for b in range(NB2):
            pltpu.make_async_copy(sbuf[b], acc.at[bidx[b]], ssem[b]).wait()

            @pl.when(g + 1 < NGRP2)
            def _(b=b):
                issue_idx(rbase + ((g + 1) * NB2 + b) * CH, b)
        return carry

    lax.fori_loop(0, NGRP2, group, 0)
    plsc.subcore_barrier()

    @pl.when(s == 0)
    def _():
        pltpu.sync_copy(acc.at[pl.ds(0, NG)], outg.at[c])


_batch_kernel = pl.kernel(
    _batch_body,
    out_type=jax.ShapeDtypeStruct((NC, NG, HF), jnp.float32),
    mesh=plsc.VectorSubcoreMesh(
        core_axis_name="c", subcore_axis_name="s", num_cores=NC,
        num_subcores=NS,
    ),
    scratch_types=_BATCH_SCRATCH,
    compiler_params=pltpu.CompilerParams(use_tc_tiling_on_sc=False),
)


RB = 2048
PRB = RB // 8
NBLK = -(-N_NODES // RB)
X8R = N_NODES * D_FEAT // 1024
PNRA = NPAD * HF // 128

_P = jax.lax.Precision.HIGHEST


def _mm(a, b):
    return jnp.dot(a, b, preferred_element_type=jnp.float32, precision=_P)


def _tc_in_body(x8, wlo, whi, bis, mws, st_ref, msg_ref):
    slo = jnp.maximum(_mm(x8[...], wlo[...]) + bis[0], 0.0)
    shi = jnp.maximum(_mm(x8[...], whi[...]) + bis[1], 0.0)
    st_ref[0] = slo
    st_ref[1] = shi
    msg_ref[0] = jnp.maximum(_mm(slo, mws[0]) + _mm(shi, mws[1]) + bis[2], 0.0)
    msg_ref[1] = jnp.maximum(_mm(slo, mws[2]) + _mm(shi, mws[3]) + bis[3], 0.0)


def _tc_round_body(agg_ref, st_ref, uws, mws, bs, st_out, msg_ref):
    alo, ahi = agg_ref[0], agg_ref[1]
    slo = st_ref[0] + jnp.maximum(_mm(alo, uws[0]) + _mm(ahi, uws[1]) + bs[0], 0.0)
    shi = st_ref[1] + jnp.maximum(_mm(alo, uws[2]) + _mm(ahi, uws[3]) + bs[1], 0.0)
    st_out[0] = slo
    st_out[1] = shi
    msg_ref[0] = jnp.maximum(_mm(slo, mws[0]) + _mm(shi, mws[1]) + bs[2], 0.0)
    msg_ref[1] = jnp.maximum(_mm(slo, mws[2]) + _mm(shi, mws[3]) + bs[3], 0.0)


def _tc_last_body(agg_ref, st_ref, uws, bs, st2_ref):
    alo, ahi = agg_ref[0], agg_ref[1]
    st2_ref[0] = st_ref[0] + jnp.maximum(
        _mm(alo, uws[0]) + _mm(ahi, uws[1]) + bs[0], 0.0)
    st2_ref[1] = st_ref[1] + jnp.maximum(
        _mm(alo, uws[2]) + _mm(ahi, uws[3]) + bs[1], 0.0)


def _tc_final_body(g0, g1, wo, bo, out_ref):
    gs = jnp.concatenate([g0[...], g1[...]], axis=-1)
    o = _mm(gs, wo[...]) + bo[...]
    out_ref[...] = jnp.concatenate([o[:, :SD], jnp.exp(o[:, SD:])], axis=-1)


def _full(shape):
    return pl.BlockSpec(shape, lambda i: (0,) * len(shape))


_plane_spec = pl.BlockSpec((NC, PRB, 128), lambda i: (0, i, 0))

_tc_in = pl.pallas_call(
    _tc_in_body,
    grid=(NBLK,),
    in_specs=[
        pl.BlockSpec((PRB, 8 * D_FEAT), lambda i: (i, 0)),
        _full((8 * D_FEAT, 128)), _full((8 * D_FEAT, 128)),
        _full((4, 1, 128)), _full((4, 128, 128)),
    ],
    out_specs=[_plane_spec, _plane_spec],
    out_shape=[
        jax.ShapeDtypeStruct((NC, PNRA, 128), jnp.float32),
        jax.ShapeDtypeStruct((NC, PNRA, 128), jnp.float32),
    ],
)

_tc_round = pl.pallas_call(
    _tc_round_body,
    grid=(NBLK,),
    in_specs=[
        _plane_spec, _plane_spec,
        _full((4, 128, 128)), _full((4, 128, 128)), _full((4, 1, 128)),
    ],
    out_specs=[_plane_spec, _plane_spec],
    out_shape=[
        jax.ShapeDtypeStruct((NC, PNRA, 128), jnp.float32),
        jax.ShapeDtypeStruct((NC, PNRA, 128), jnp.float32),
    ],
)

_tc_last = pl.pallas_call(
    _tc_last_body,
    grid=(NBLK,),
    in_specs=[
        _plane_spec, _plane_spec,
        _full((4, 128, 128)), _full((2, 1, 128)),
    ],
    out_specs=_plane_spec,
    out_shape=jax.ShapeDtypeStruct((NC, PNRA, 128), jnp.float32),
)

_tc_final = pl.pallas_call(
    _tc_final_body,
    grid=(1,),
    in_specs=[
        _full((NG, HF)), _full((NG, HF)),
        _full((SD, 2 * SD)), _full((1, 2 * SD)),
    ],
    out_specs=_full((NG, 2 * SD)),
    out_shape=jax.ShapeDtypeStruct((NG, 2 * SD), jnp.float32),
)


def _bd8(block):
    return jnp.kron(jnp.eye(8, dtype=block.dtype), block)


def _tile8(vec):
    return jnp.tile(vec, 8).reshape(1, 128)


@jax.jit
def kernel(x, edge_index, batch, W_in, b_in, msg_W, msg_b, upd_W, upd_b,
           W_out, b_out):
    src = edge_index[0].astype(jnp.int32)
    dst = edge_index[1].astype(jnp.int32)
    pad = EP - N_EDGES
    zpad = jnp.zeros((pad,), jnp.int32)
    srcs = jnp.concatenate([src, zpad, src + NPAD, zpad])
    dstp = jnp.concatenate([dst, jnp.full((pad,), N_NODES, jnp.int32)])
    bpad = jnp.concatenate(
        [batch.astype(jnp.int32), jnp.full((NPAD - N_NODES,), NG, jnp.int32)])

    x8 = x.reshape(X8R, 8 * D_FEAT)
    wlo = _bd8(W_in[:, :HF])
    whi = _bd8(W_in[:, HF:])

    def quad(W):
        return jnp.stack([_bd8(W[:HF, :HF]), _bd8(W[HF:, :HF]),
                          _bd8(W[:HF, HF:]), _bd8(W[HF:, HF:])])

    mwq = [quad(msg_W[r]) for r in range(ROUNDS_N)]
    uwq = [quad(upd_W[r]) for r in range(ROUNDS_N)]
    mb2 = [(_tile8(msg_b[r, :HF]), _tile8(msg_b[r, HF:]))
           for r in range(ROUNDS_N)]
    ub2 = [(_tile8(upd_b[r, :HF]), _tile8(upd_b[r, HF:]))
           for r in range(ROUNDS_N)]

    bis = jnp.stack([_tile8(b_in[:HF]), _tile8(b_in[HF:]),
                     mb2[0][0], mb2[0][1]])
    stp, msgp = _tc_in(x8, wlo, whi, bis, mwq[0])
    st2 = None
    for r in range(ROUNDS_N):
        agg = _edge_kernel(msgp.reshape(NC * NPAD, HF), srcs, dstp)
        aggp = agg.reshape(NC, PNRA, 128)
        if r < ROUNDS_N - 1:
            bs = jnp.stack([ub2[r][0], ub2[r][1],
                            mb2[r + 1][0], mb2[r + 1][1]])
            stp, msgp = _tc_round(aggp, stp, uwq[r], mwq[r + 1], bs)
        else:
            bs = jnp.stack([ub2[r][0], ub2[r][1]])
            st2 = _tc_last(aggp, stp, uwq[r], bs)
    gs2 = _batch_kernel(st2.reshape(NC, NPAD, HF), bpad)
    out = _tc_final(gs2[0], gs2[1], W_out, b_out.reshape(1, 2 * SD))
    return out


# speedup vs baseline: 19.5440x; 1.0856x over previous
import jax
import jax.numpy as jnp
from jax import lax
from jax.experimental import pallas as pl
from jax.experimental.pallas import tpu as pltpu
from jax.experimental.pallas import tpu_sc as plsc

N_NODES = 100000
N_EDGES = 1600000
D_FEAT = 128
SD = 32
HF = 16
NG = 512
ROUNDS_N = 4

NC = 2
NS = 16
CH = 128
NB = 8

EPG = NS * CH * NB
NGRP = -(-N_EDGES // EPG)
EP = NGRP * EPG
EPT = EP // NS
NPAD = 100352
ZPT = NPAD // NS
ZB = 448

NB2 = 7
NGRP2 = 7
GPAD = 520

_EDGE_SCRATCH = (
    [pltpu.VMEM_SHARED((NPAD, HF), jnp.float32)]
    + [pltpu.VMEM((ZB, HF), jnp.float32)]
    + [pltpu.VMEM((CH,), jnp.int32) for _ in range(NB)]
    + [pltpu.VMEM((CH,), jnp.int32) for _ in range(NB)]
    + [pltpu.VMEM((CH, HF), jnp.float32) for _ in range(NB)]
    + [pltpu.SemaphoreType.DMA]
    + [pltpu.SemaphoreType.DMA for _ in range(3 * NB)]
)


def _edge_body(msg2, srcs, dstp, out, *scr):
    acc = scr[0]
    zbuf = scr[1]
    sidx = scr[2:2 + NB]
    didx = scr[2 + NB:2 + 2 * NB]
    gbuf = scr[2 + 2 * NB:2 + 3 * NB]
    zsem = scr[2 + 3 * NB]
    isem = scr[3 + 3 * NB:3 + 4 * NB]
    gsem = scr[3 + 4 * NB:3 + 5 * NB]
    ssem = scr[3 + 5 * NB:3 + 6 * NB]

    c = lax.axis_index("c")
    s = lax.axis_index("s")

    def zfill(i, carry):
        zbuf[i, :] = jnp.zeros((HF,), jnp.float32)
        return carry
    lax.fori_loop(0, ZB, zfill, 0)
    z0 = s * ZPT
    for k in range(ZPT // ZB):
        pltpu.async_copy(zbuf, acc.at[pl.ds(z0 + k * ZB, ZB)], zsem)
    for k in range(ZPT // ZB):
        pltpu.make_async_copy(zbuf, acc.at[pl.ds(z0 + k * ZB, ZB)], zsem).wait()
    plsc.subcore_barrier()

    ebase = s * EPT

    def issue_idx(off, b):
        pltpu.async_copy(srcs.at[pl.ds(c * EP + off, CH)], sidx[b], isem[b])
        pltpu.async_copy(dstp.at[pl.ds(off, CH)], didx[b], isem[b])

    for b in range(NB):
        issue_idx(ebase + b * CH, b)

    def group(g, carry):
        for b in range(NB):
            pltpu.make_async_copy(srcs.at[pl.ds(0, CH)], sidx[b], isem[b]).wait()
            pltpu.make_async_copy(dstp.at[pl.ds(0, CH)], didx[b], isem[b]).wait()
            pltpu.async_copy(msg2.at[sidx[b]], gbuf[b], gsem[b])
        for b in range(NB):
            pltpu.make_async_copy(msg2.at[sidx[b]], gbuf[b], gsem[b]).wait()
            pltpu.async_copy(gbuf[b], acc.at[didx[b]], ssem[b], add=True)
        for b in range(NB):
            pltpu.make_async_copy(gbuf[b], acc.at[didx[b]], ssem[b]).wait()

            @pl.when(g + 1 < NGRP)
            def _(b=b):
                issue_idx(ebase + ((g + 1) * NB + b) * CH, b)
        return carry

    lax.fori_loop(0, NGRP, group, 0)
    plsc.subcore_barrier()

    pltpu.sync_copy(acc.at[pl.ds(z0, ZPT)], out.at[c, pl.ds(z0, ZPT)])


_edge_kernel = pl.kernel(
    _edge_body,
    out_type=jax.ShapeDtypeStruct((NC, NPAD, HF), jnp.float32),
    mesh=plsc.VectorSubcoreMesh(
        core_axis_name="c", subcore_axis_name="s", num_cores=NC,
        num_subcores=NS,
    ),
    scratch_types=_EDGE_SCRATCH,
    compiler_params=pltpu.CompilerParams(use_tc_tiling_on_sc=False),
)


_BATCH_SCRATCH = (
    [pltpu.VMEM_SHARED((GPAD, HF), jnp.float32)]
    + [pltpu.VMEM((GPAD, HF), jnp.float32)]
    + [pltpu.VMEM((CH,), jnp.int32) for _ in range(NB2)]
    + [pltpu.VMEM((CH, HF), jnp.float32) for _ in range(NB2)]
    + [pltpu.SemaphoreType.DMA]
    + [pltpu.SemaphoreType.DMA for _ in range(2 * NB2)]
)


def _batch_body(st2, bidx_hbm, outg, *scr):
    acc = scr[0]
    zbuf = scr[1]
    bidx = scr[2:2 + NB2]
    sbuf = scr[2 + NB2:2 + 2 * NB2]
    zsem = scr[2 + 2 * NB2]
    isem = scr[3 + 2 * NB2:3 + 3 * NB2]
    ssem = scr[3 + 3 * NB2:3 + 4 * NB2]

    c = lax.axis_index("c")
    s = lax.axis_index("s")

    @pl.when(s == 0)
    def _():
        def zfill(i, carry):
            zbuf[i, :] = jnp.zeros((HF,), jnp.float32)
            return carry
        lax.fori_loop(0, GPAD, zfill, 0)
        pltpu.async_copy(zbuf, acc, zsem)
        pltpu.make_async_copy(zbuf, acc, zsem).wait()
    plsc.subcore_barrier()

    rbase = s * ZPT

    def issue_idx(off, b):
        pltpu.async_copy(st2.at[c, pl.ds(off, CH)], sbuf[b], isem[b])
        pltpu.async_copy(bidx_hbm.at[pl.ds(off, CH)], bidx[b], isem[b])

    for b in range(NB2):
        issue_idx(rbase + b * CH, b)

    def group(g, carry):
        for b in range(NB2):
            pltpu.make_async_copy(st2.at[c, pl.ds(0, CH)], sbuf[b], isem[b]).wait()
            pltpu.make_async_copy(bidx_hbm.at[pl.ds(0, CH)], bidx[b], isem[b]).wait()
            pltpu.async_copy(sbuf[b], acc.at[bidx[b]], ssem[b], add=True)
        for b in range(NB2):
            pltpu.make_async_copy(sbuf[b], acc.at[bidx[b]], ssem[b]).wait()

            @pl.when(g + 1 < NGRP2)
            def _(b=b):
                issue_idx(rbase + ((g + 1) * NB2 + b) * CH, b)
        return carry

    lax.fori_loop(0, NGRP2, group, 0)
    plsc.subcore_barrier()

    @pl.when(s == 0)
    def _():
        pltpu.sync_copy(acc.at[pl.ds(0, NG)], outg.at[c])


_batch_kernel = pl.kernel(
    _batch_body,
    out_type=jax.ShapeDtypeStruct((NC, NG, HF), jnp.float32),
    mesh=plsc.VectorSubcoreMesh(
        core_axis_name="c", subcore_axis_name="s", num_cores=NC,
        num_subcores=NS,
    ),
    scratch_types=_BATCH_SCRATCH,
    compiler_params=pltpu.CompilerParams(use_tc_tiling_on_sc=False),
)


RB = 2048
PRB = RB // 8
NBLK = -(-N_NODES // RB)
X8R = N_NODES * D_FEAT // 1024
PNRA = NPAD * HF // 128

def _mm(a, b):
    return jnp.dot(a, b, preferred_element_type=jnp.float32)


def _tc_in_body(x8, wlo, whi, bis, mws, st_ref, msg_ref):
    slo = jnp.maximum(_mm(x8[...], wlo[...]) + bis[0], 0.0)
    shi = jnp.maximum(_mm(x8[...], whi[...]) + bis[1], 0.0)
    st_ref[0] = slo
    st_ref[1] = shi
    msg_ref[0] = jnp.maximum(_mm(slo, mws[0]) + _mm(shi, mws[1]) + bis[2], 0.0)
    msg_ref[1] = jnp.maximum(_mm(slo, mws[2]) + _mm(shi, mws[3]) + bis[3], 0.0)


def _tc_round_body(agg_ref, st_ref, uws, mws, bs, st_out, msg_ref):
    alo, ahi = agg_ref[0], agg_ref[1]
    slo = st_ref[0] + jnp.maximum(_mm(alo, uws[0]) + _mm(ahi, uws[1]) + bs[0], 0.0)
    shi = st_ref[1] + jnp.maximum(_mm(alo, uws[2]) + _mm(ahi, uws[3]) + bs[1], 0.0)
    st_out[0] = slo
    st_out[1] = shi
    msg_ref[0] = jnp.maximum(_mm(slo, mws[0]) + _mm(shi, mws[1]) + bs[2], 0.0)
    msg_ref[1] = jnp.maximum(_mm(slo, mws[2]) + _mm(shi, mws[3]) + bs[3], 0.0)


def _tc_last_body(agg_ref, st_ref, uws, bs, st2_ref):
    alo, ahi = agg_ref[0], agg_ref[1]
    st2_ref[0] = st_ref[0] + jnp.maximum(
        _mm(alo, uws[0]) + _mm(ahi, uws[1]) + bs[0], 0.0)
    st2_ref[1] = st_ref[1] + jnp.maximum(
        _mm(alo, uws[2]) + _mm(ahi, uws[3]) + bs[1], 0.0)


def _tc_final_body(g0, g1, wo, bo, out_ref):
    gs = jnp.concatenate([g0[...], g1[...]], axis=-1)
    o = _mm(gs, wo[...]) + bo[...]
    out_ref[...] = jnp.concatenate([o[:, :SD], jnp.exp(o[:, SD:])], axis=-1)


def _full(shape):
    return pl.BlockSpec(shape, lambda i: (0,) * len(shape))


_plane_spec = pl.BlockSpec((NC, PRB, 128), lambda i: (0, i, 0))

_tc_in = pl.pallas_call(
    _tc_in_body,
    grid=(NBLK,),
    in_specs=[
        pl.BlockSpec((PRB, 8 * D_FEAT), lambda i: (i, 0)),
        _full((8 * D_FEAT, 128)), _full((8 * D_FEAT, 128)),
        _full((4, 1, 128)), _full((4, 128, 128)),
    ],
    out_specs=[_plane_spec, _plane_spec],
    out_shape=[
        jax.ShapeDtypeStruct((NC, PNRA, 128), jnp.float32),
        jax.ShapeDtypeStruct((NC, PNRA, 128), jnp.float32),
    ],
)

_tc_round = pl.pallas_call(
    _tc_round_body,
    grid=(NBLK,),
    in_specs=[
        _plane_spec, _plane_spec,
        _full((4, 128, 128)), _full((4, 128, 128)), _full((4, 1, 128)),
    ],
    out_specs=[_plane_spec, _plane_spec],
    out_shape=[
        jax.ShapeDtypeStruct((NC, PNRA, 128), jnp.float32),
        jax.ShapeDtypeStruct((NC, PNRA, 128), jnp.float32),
    ],
)

_tc_last = pl.pallas_call(
    _tc_last_body,
    grid=(NBLK,),
    in_specs=[
        _plane_spec, _plane_spec,
        _full((4, 128, 128)), _full((2, 1, 128)),
    ],
    out_specs=_plane_spec,
    out_shape=jax.ShapeDtypeStruct((NC, PNRA, 128), jnp.float32),
)

_tc_final = pl.pallas_call(
    _tc_final_body,
    grid=(1,),
    in_specs=[
        _full((NG, HF)), _full((NG, HF)),
        _full((SD, 2 * SD)), _full((1, 2 * SD)),
    ],
    out_specs=_full((NG, 2 * SD)),
    out_shape=jax.ShapeDtypeStruct((NG, 2 * SD), jnp.float32),
)


def _bd_batch(blocks):
    n, a, b = blocks.shape
    t = jnp.tile(blocks, (1, 8, 8))
    r = jax.lax.broadcasted_iota(jnp.int32, (8 * a, 8 * b), 0) // a
    c = jax.lax.broadcasted_iota(jnp.int32, (8 * a, 8 * b), 1) // b
    return t * (r == c).astype(blocks.dtype)[None]


def _tile8(vec):
    return jnp.tile(vec, 8).reshape(1, 128)


@jax.jit
def kernel(x, edge_index, batch, W_in, b_in, msg_W, msg_b, upd_W, upd_b,
           W_out, b_out):
    src = edge_index[0].astype(jnp.int32)
    dst = edge_index[1].astype(jnp.int32)
    pad = EP - N_EDGES
    zpad = jnp.zeros((pad,), jnp.int32)
    srcs = jnp.concatenate([src, zpad, src + NPAD, zpad])
    dstp = jnp.concatenate([dst, jnp.full((pad,), N_NODES, jnp.int32)])
    bpad = jnp.concatenate(
        [batch.astype(jnp.int32), jnp.full((NPAD - N_NODES,), NG, jnp.int32)])

    x8 = x.reshape(X8R, 8 * D_FEAT)
    win2 = _bd_batch(jnp.stack([W_in[:, :HF], W_in[:, HF:]]))
    wlo, whi = win2[0], win2[1]

    def quads(W):
        return jnp.stack([W[:, :HF, :HF], W[:, HF:, :HF],
                          W[:, :HF, HF:], W[:, HF:, HF:]], axis=1)

    allq = jnp.concatenate([quads(msg_W), quads(upd_W)]).reshape(-1, HF, HF)
    bd = _bd_batch(allq).reshape(2, ROUNDS_N, 4, 128, 128)
    mwq = [bd[0, r] for r in range(ROUNDS_N)]
    uwq = [bd[1, r] for r in range(ROUNDS_N)]
    mb2 = [(_tile8(msg_b[r, :HF]), _tile8(msg_b[r, HF:]))
           for r in range(ROUNDS_N)]
    ub2 = [(_tile8(upd_b[r, :HF]), _tile8(upd_b[r, HF:]))
           for r in range(ROUNDS_N)]

    bis = jnp.stack([_tile8(b_in[:HF]), _tile8(b_in[HF:]),
                     mb2[0][0], mb2[0][1]])
    stp, msgp = _tc_in(x8, wlo, whi, bis, mwq[0])
    st2 = None
    for r in range(ROUNDS_N):
        agg = _edge_kernel(msgp.reshape(NC * NPAD, HF), srcs, dstp)
        aggp = agg.reshape(NC, PNRA, 128)
        if r < ROUNDS_N - 1:
            bs = jnp.stack([ub2[r][0], ub2[r][1],
                            mb2[r + 1][0], mb2[r + 1][1]])
            stp, msgp = _tc_round(aggp, stp, uwq[r], mwq[r + 1], bs)
        else:
            bs = jnp.stack([ub2[r][0], ub2[r][1]])
            st2 = _tc_last(aggp, stp, uwq[r], bs)
    gs2 = _batch_kernel(st2.reshape(NC, NPAD, HF), bpad)
    out = _tc_final(gs2[0], gs2[1], W_out, b_out.reshape(1, 2 * SD))
    return out
